# Initial kernel scaffold; baseline (speedup 1.0000x reference)
import jax
import jax.numpy as jnp
from jax.experimental import pallas as pl


def kernel(node_seq, fin_seq, nfin_seq, mda_seq, seq_len, node_table, fin_table, nfin_table, bert_table, bn_gamma, bn_beta, W1, b1, W2, b2):
    raise NotImplementedError("write your pallas kernel here")


# speedup vs baseline: 1.3473x; 1.3473x over previous
import functools

import jax
import jax.numpy as jnp
from jax import lax
from jax.experimental import pallas as pl
from jax.experimental.pallas import tpu as pltpu
from jax.experimental.pallas import tpu_sc as plsc

B = 16384
D_NODE, D_FIN, D_NFIN, D_BERT = 128, 84, 11, 768
PAD = 128
D_HID = 1982
NC, NS = 2, 16
NW = NC * NS
BPW = B // NW
CH = 128
CHB = 64
EPS = 1e-4


def _gather_body(node_idx, fin_idx, nfin_idx, mda_idx,
                 node_table, fin_table, nfin_table, bert_table,
                 out_node, out_fin, out_nfin, out_bert,
                 idx_n, idx_f, idx_nf, idx_m,
                 buf_n, buf_f, buf_nf, buf_b, sem):
    wid = lax.axis_index("s") * NC + lax.axis_index("c")
    base = wid * BPW
    pltpu.sync_copy(node_idx.at[pl.ds(base, BPW)], idx_n)
    pltpu.sync_copy(fin_idx.at[pl.ds(base, BPW)], idx_f)
    pltpu.sync_copy(nfin_idx.at[pl.ds(base, BPW)], idx_nf)
    pltpu.sync_copy(mda_idx.at[pl.ds(base, BPW)], idx_m)
    for c in range(BPW // CH):
        pltpu.async_copy(
            node_table.at[idx_n.at[pl.ds(c * CH, CH)]], buf_n, sem).wait()
        pltpu.sync_copy(buf_n, out_node.at[pl.ds(base + c * CH, CH)])
        pltpu.async_copy(
            fin_table.at[idx_f.at[pl.ds(c * CH, CH)]], buf_f, sem).wait()
        pltpu.sync_copy(buf_f, out_fin.at[pl.ds(base + c * CH, CH)])
        pltpu.async_copy(
            nfin_table.at[idx_nf.at[pl.ds(c * CH, CH)]], buf_nf, sem).wait()
        pltpu.sync_copy(buf_nf, out_nfin.at[pl.ds(base + c * CH, CH)])
    for c in range(BPW // CHB):
        pltpu.async_copy(
            bert_table.at[idx_m.at[pl.ds(c * CHB, CHB)]], buf_b, sem).wait()
        pltpu.sync_copy(buf_b, out_bert.at[pl.ds(base + c * CHB, CHB)])


@functools.cache
def _build_gather():
    mesh = plsc.VectorSubcoreMesh(core_axis_name="c", subcore_axis_name="s",
                                  num_cores=NC, num_subcores=NS)
    return pl.kernel(
        _gather_body,
        out_type=(
            jax.ShapeDtypeStruct((B, D_NODE), jnp.float32),
            jax.ShapeDtypeStruct((B, PAD), jnp.float32),
            jax.ShapeDtypeStruct((B, PAD), jnp.float32),
            jax.ShapeDtypeStruct((B, D_BERT), jnp.float32),
        ),
        mesh=mesh,
        scratch_types=[
            pltpu.VMEM((BPW,), jnp.int32),
            pltpu.VMEM((BPW,), jnp.int32),
            pltpu.VMEM((BPW,), jnp.int32),
            pltpu.VMEM((BPW,), jnp.int32),
            pltpu.VMEM((CH, D_NODE), jnp.float32),
            pltpu.VMEM((CH, PAD), jnp.float32),
            pltpu.VMEM((CH, PAD), jnp.float32),
            pltpu.VMEM((CHB, D_BERT), jnp.float32),
            pltpu.SemaphoreType.DMA,
        ],
    )


ROWS = 512
WIDTHS = (D_NODE, PAD, PAD, D_BERT)


def _stats_kernel(xn, xf, xnf, xb, on, of, onf, ob):
    @pl.when(pl.program_id(0) == 0)
    def _():
        on[...] = jnp.zeros_like(on)
        of[...] = jnp.zeros_like(of)
        onf[...] = jnp.zeros_like(onf)
        ob[...] = jnp.zeros_like(ob)

    for x, o in ((xn, on), (xf, of), (xnf, onf), (xb, ob)):
        v = x[...]
        s = jnp.sum(v, axis=0, keepdims=True)
        q = jnp.sum(v * v, axis=0, keepdims=True)
        o[...] += jnp.concatenate([s, q], axis=0)


def _stats(xn, xf, xnf, xb):
    grid = (B // ROWS,)
    blk = lambda w: pl.BlockSpec((ROWS, w), lambda i: (i, 0))
    out_blk = lambda w: pl.BlockSpec((2, w), lambda i: (0, 0))
    return pl.pallas_call(
        _stats_kernel,
        grid=grid,
        in_specs=[blk(w) for w in WIDTHS],
        out_specs=tuple(out_blk(w) for w in WIDTHS),
        out_shape=tuple(
            jax.ShapeDtypeStruct((2, w), jnp.float32) for w in WIDTHS),
    )(xn, xf, xnf, xb)


def _matvec_kernel(xn, xf, xnf, xb,
                   stn, stf, stnf, stb,
                   gn, gf, gnf, gb,
                   bn, bf, bnf, bb,
                   w1n, w1f, w1nf, w1b,
                   w2, b1, b2,
                   out,
                   s_n, s_f, s_nf, s_b,
                   wc_n, wc_f, wc_nf, wc_b, cst):
    @pl.when(pl.program_id(0) == 0)
    def _():
        const = jnp.dot(b1[...], w2[...],
                        preferred_element_type=jnp.float32) + b2[...]
        for st, g, bt, w1p, s_scr, wc_scr in (
                (stn, gn, bn, w1n, s_n, wc_n),
                (stf, gf, bf, w1f, s_f, wc_f),
                (stnf, gnf, bnf, w1nf, s_nf, wc_nf),
                (stb, gb, bb, w1b, s_b, wc_b)):
            mean = st[0:1, :] * (1.0 / B)
            ex2 = st[1:2, :] * (1.0 / B)
            var = ex2 - mean * mean
            inv = lax.rsqrt(var + EPS)
            s = g[...] * inv
            c = bt[...] - mean * s
            wc = jnp.dot(w1p[...], w2[...],
                         preferred_element_type=jnp.float32)
            s_scr[...] = s
            wc_scr[...] = wc
            const = const + jnp.dot(c, wc,
                                    preferred_element_type=jnp.float32)
        cst[...] = const

    acc = jnp.broadcast_to(cst[...], (ROWS, 2))
    for x, s_scr, wc_scr in ((xn, s_n, wc_n), (xf, s_f, wc_f),
                             (xnf, s_nf, wc_nf), (xb, s_b, wc_b)):
        acc = acc + jnp.dot(x[...] * s_scr[...], wc_scr[...],
                            preferred_element_type=jnp.float32)
    out[...] = acc


def _matvec(xn, xf, xnf, xb, stats, gamma_p, beta_p, w1_p, W2, b1, b2):
    grid = (B // ROWS,)
    blk = lambda w: pl.BlockSpec((ROWS, w), lambda i: (i, 0))
    full = lambda a: pl.BlockSpec(a.shape, lambda i: tuple(0 for _ in a.shape))
    in_specs = (
        [blk(w) for w in WIDTHS]
        + [full(s) for s in stats]
        + [full(g) for g in gamma_p]
        + [full(b) for b in beta_p]
        + [full(w) for w in w1_p]
        + [full(W2), full(b1), full(b2)]
    )
    scratch = ([pltpu.VMEM((1, w), jnp.float32) for w in WIDTHS]
               + [pltpu.VMEM((w, 2), jnp.float32) for w in WIDTHS]
               + [pltpu.VMEM((1, 2), jnp.float32)])
    return pl.pallas_call(
        _matvec_kernel,
        grid=grid,
        in_specs=in_specs,
        out_specs=pl.BlockSpec((ROWS, 2), lambda i: (i, 0)),
        out_shape=jax.ShapeDtypeStruct((B, 2), jnp.float32),
        scratch_shapes=scratch,
    )(xn, xf, xnf, xb, *stats, *gamma_p, *beta_p, *w1_p, W2, b1, b2)


def kernel(node_seq, fin_seq, nfin_seq, mda_seq, seq_len,
           node_table, fin_table, nfin_table, bert_table,
           bn_gamma, bn_beta, W1, b1, W2, b2):
    ni = node_seq.reshape(B).astype(jnp.int32)
    fi = fin_seq.reshape(B).astype(jnp.int32)
    nfi = nfin_seq.reshape(B).astype(jnp.int32)
    mi = mda_seq.reshape(B).astype(jnp.int32)

    fin_pad = jnp.pad(fin_table, ((0, 0), (0, PAD - D_FIN)))
    nfin_pad = jnp.pad(nfin_table, ((0, 0), (0, PAD - D_NFIN)))

    xn, xf, xnf, xb = _build_gather()(ni, fi, nfi, mi,
                                      node_table, fin_pad, nfin_pad,
                                      bert_table)
    stats = _stats(xn, xf, xnf, xb)

    splits = (0, D_NODE, D_NODE + D_FIN, D_NODE + D_FIN + D_NFIN,
              D_NODE + D_FIN + D_NFIN + D_BERT)
    pieces = lambda a: tuple(a[splits[i]:splits[i + 1]] for i in range(4))
    padw = lambda p, w: jnp.pad(p, ((0, w - p.shape[0]),) +
                                ((0, 0),) * (p.ndim - 1))
    gamma_p = tuple(padw(p, w).reshape(1, w)
                    for p, w in zip(pieces(bn_gamma), WIDTHS))
    beta_p = tuple(padw(p, w).reshape(1, w)
                   for p, w in zip(pieces(bn_beta), WIDTHS))
    w1_p = tuple(padw(p, w) for p, w in zip(pieces(W1), WIDTHS))

    return _matvec(xn, xf, xnf, xb, stats, gamma_p, beta_p, w1_p,
                   W2, b1.reshape(1, -1), b2.reshape(1, -1))
